# Initial kernel scaffold; baseline (speedup 1.0000x reference)
import jax
import jax.numpy as jnp
from jax.experimental import pallas as pl


def kernel(x, pos, batch, batch_size, c1_W1, c1_b1, c1_g1, c1_be1, c1_W2, c1_b2, c2_W1, c2_b1, c2_g1, c2_be1, c2_W2, c2_b2, c3_W1, c3_b1, c3_g1, c3_be1, c3_W2, c3_b2, m_W1, m_b1, m_W2, m_b2, m_W3, m_b3):
    raise NotImplementedError("write your pallas kernel here")


# speedup vs baseline: 6.8803x; 6.8803x over previous
import functools

import jax
import jax.numpy as jnp
from jax import lax
from jax.experimental import pallas as pl
from jax.experimental.pallas import tpu as pltpu
from jax.experimental.pallas import tpu_sc as plsc

B = 8
P = 1024
N = B * P
K = 30
KPAD = 32
OUT = 13
EPS = 1e-5
SPLITS = 2
RB = P // SPLITS
E_TOT = float(N * K)
BF = jnp.bfloat16

NC = 2
NS = 16
NW = NC * NS
B_PER_W = (N * K) // NW
CH = 128
NCH = B_PER_W // CH
NBUF = 6


def _knn_body(xr_ref, xc_ref, idx_ref):
    b = pl.program_id(0)
    xr = xr_ref[...]
    xc = xc_ref[...]
    sqr = jnp.sum(xr * xr, axis=1, keepdims=True)
    sqc = jnp.sum(xc * xc, axis=1)
    gram = lax.dot_general(xr.astype(BF), xc.astype(BF),
                           (((1,), (1,)), ((), ())),
                           preferred_element_type=jnp.float32)
    d = (sqr - 2.0 * gram) + sqc[None, :]

    iota = lax.broadcasted_iota(jnp.int32, (RB, P), 1)
    for k in range(K):
        m = jnp.min(d, axis=1, keepdims=True)
        cand = jnp.where(d == m, iota, P)
        j = jnp.min(cand, axis=1, keepdims=True)
        d = jnp.where(iota == j, jnp.inf, d)
        idx_ref[:, k:k + 1] = j + b * P


def _knn(x):
    c = x.shape[1]
    return pl.pallas_call(
        _knn_body,
        grid=(B, SPLITS),
        in_specs=[
            pl.BlockSpec((RB, c), lambda b, h: (b * SPLITS + h, 0)),
            pl.BlockSpec((P, c), lambda b, h: (b, 0)),
        ],
        out_specs=pl.BlockSpec((RB, KPAD), lambda b, h: (b * SPLITS + h, 0)),
        out_shape=jax.ShapeDtypeStruct((N, KPAD), jnp.int32),
    )(x, x)


def _sc_gather_body(cw, table_ref, idx_ref, out_ref, idx_v, bufs, *sems):
    gsem = sems[:NBUF]
    ssem = sems[NBUF:]
    wid = lax.axis_index("s") * NC + lax.axis_index("c")
    base = wid * B_PER_W
    pltpu.sync_copy(idx_ref.at[pl.ds(base, B_PER_W)], idx_v)
    for db in range(NBUF):
        pltpu.async_copy(table_ref.at[idx_v.at[pl.ds(db * CH, CH)]],
                         bufs.at[db], gsem[db])

    def outer(it, carry):
        ci0 = it * NBUF
        for db in range(NBUF):
            ci = ci0 + db
            pltpu.make_async_copy(table_ref.at[pl.ds(0, CH)],
                                  bufs.at[db], gsem[db]).wait()
            pltpu.async_copy(bufs.at[db],
                             out_ref.at[pl.ds(base + ci * CH, CH)], ssem[db])
            nci = ci + NBUF

            @pl.when(nci < NCH)
            def _():
                pltpu.make_async_copy(bufs.at[db],
                                      out_ref.at[pl.ds(base, CH)],
                                      ssem[db]).wait()
                pltpu.async_copy(
                    table_ref.at[idx_v.at[pl.ds(nci * CH, CH)]],
                    bufs.at[db], gsem[db])
        return carry

    lax.fori_loop(0, NCH // NBUF, outer, 0)
    for db in range(NBUF):
        pltpu.make_async_copy(bufs.at[db],
                              out_ref.at[pl.ds(base, CH)], ssem[db]).wait()


def _sc_gather(table, idx_flat):
    cw = table.shape[1]
    kfn = functools.partial(
        pl.kernel,
        mesh=plsc.VectorSubcoreMesh(core_axis_name="c", subcore_axis_name="s"),
        compiler_params=pltpu.CompilerParams(use_tc_tiling_on_sc=False),
        out_type=jax.ShapeDtypeStruct((N * K, cw), jnp.float32),
        scratch_types=(
            [pltpu.VMEM((B_PER_W,), jnp.int32),
             pltpu.VMEM((NBUF, CH, cw), jnp.float32)]
            + [pltpu.SemaphoreType.DMA] * (2 * NBUF)
        ),
    )(functools.partial(_sc_gather_body, cw))
    return kfn(table, idx_flat)


PBB = 512


def _edge_h(xg_ref, xi_ref, w1t16, w1b16, b1):
    xi = xi_ref[...]
    hxi = jnp.dot(xi.astype(BF), w1t16,
                  preferred_element_type=jnp.float32)
    for k in range(K):
        diff = xg_ref[:, k, :] - xi
        hk = hxi + jnp.dot(diff.astype(BF), w1b16,
                           preferred_element_type=jnp.float32)
        yield hk + b1


def _moments_body(xg_ref, xi_ref, w1t_ref, w1b_ref, b1_ref, stats_ref):
    w1t16 = w1t_ref[...].astype(BF)
    w1b16 = w1b_ref[...].astype(BF)
    b1 = b1_ref[...]
    s1 = jnp.zeros((1, 64), jnp.float32)
    s2 = jnp.zeros((1, 64), jnp.float32)
    for h in _edge_h(xg_ref, xi_ref, w1t16, w1b16, b1):
        s1 = s1 + jnp.sum(h, axis=0, keepdims=True)
        s2 = s2 + jnp.sum(h * h, axis=0, keepdims=True)

    @pl.when(pl.program_id(0) == 0)
    def _():
        stats_ref[...] = jnp.zeros_like(stats_ref)

    stats_ref[0:1, :] += s1
    stats_ref[1:2, :] += s2


def _moments(xg3, xpad, w1t, w1b, b1):
    cw = xpad.shape[1]
    return pl.pallas_call(
        _moments_body,
        grid=(N // PBB,),
        in_specs=[
            pl.BlockSpec((PBB, K, cw), lambda g: (g, 0, 0)),
            pl.BlockSpec((PBB, cw), lambda g: (g, 0)),
            pl.BlockSpec((cw, 64), lambda g: (0, 0)),
            pl.BlockSpec((cw, 64), lambda g: (0, 0)),
            pl.BlockSpec((1, 64), lambda g: (0, 0)),
        ],
        out_specs=pl.BlockSpec((8, 64), lambda g: (0, 0)),
        out_shape=jax.ShapeDtypeStruct((8, 64), jnp.float32),
    )(xg3, xpad, w1t, w1b, b1)


def _edge_out_body(xg_ref, xi_ref, w1t_ref, w1b_ref, b1_ref,
                   g1_ref, mean_ref, rvec_ref, be_ref, w2_ref, b2_ref,
                   out_ref):
    w1t16 = w1t_ref[...].astype(BF)
    w1b16 = w1b_ref[...].astype(BF)
    w216 = w2_ref[...].astype(BF)
    b1 = b1_ref[...]
    g1 = g1_ref[...]
    mean = mean_ref[...]
    rvec = rvec_ref[...]
    be = be_ref[...]
    acc = jnp.full((PBB, 64), -jnp.inf, jnp.float32)
    for h in _edge_h(xg_ref, xi_ref, w1t16, w1b16, b1):
        t = g1 * (h - mean)
        t = t * rvec
        t = t + be
        t = jnp.maximum(t, 0.0)
        o = jnp.dot(t.astype(BF), w216, preferred_element_type=jnp.float32)
        acc = jnp.maximum(acc, o)
    out_ref[...] = acc + b2_ref[...]


def _edge_out(xg3, xpad, w1t, w1b, b1, g1, mean, rvec, be, w2, b2):
    cw = xpad.shape[1]
    return pl.pallas_call(
        _edge_out_body,
        grid=(N // PBB,),
        in_specs=[
            pl.BlockSpec((PBB, K, cw), lambda g: (g, 0, 0)),
            pl.BlockSpec((PBB, cw), lambda g: (g, 0)),
            pl.BlockSpec((cw, 64), lambda g: (0, 0)),
            pl.BlockSpec((cw, 64), lambda g: (0, 0)),
            pl.BlockSpec((1, 64), lambda g: (0, 0)),
            pl.BlockSpec((1, 64), lambda g: (0, 0)),
            pl.BlockSpec((1, 64), lambda g: (0, 0)),
            pl.BlockSpec((1, 64), lambda g: (0, 0)),
            pl.BlockSpec((1, 64), lambda g: (0, 0)),
            pl.BlockSpec((64, 64), lambda g: (0, 0)),
            pl.BlockSpec((1, 64), lambda g: (0, 0)),
        ],
        out_specs=pl.BlockSpec((PBB, 64), lambda g: (g, 0)),
        out_shape=jax.ShapeDtypeStruct((N, 64), jnp.float32),
    )(xg3, xpad, w1t, w1b, b1, g1, mean, rvec, be, w2, b2)


def _edge_conv(x, w1, b1, g1, be1, w2, b2):
    c = x.shape[1]
    cw = 16 if c < 16 else c
    if cw != c:
        xpad = jnp.zeros((N, cw), jnp.float32).at[:, :c].set(x)
        w1t = jnp.zeros((cw, 64), jnp.float32).at[:c].set(w1[:c])
        w1b = jnp.zeros((cw, 64), jnp.float32).at[:c].set(w1[c:])
    else:
        xpad = x
        w1t = w1[:c]
        w1b = w1[c:]
    idx = _knn(x)
    idx_flat = idx[:, :K].reshape(N * K)
    xg3 = _sc_gather(xpad, idx_flat).reshape(N, K, cw)
    stats = _moments(xg3, xpad, w1t, w1b, b1.reshape(1, 64))
    mean = stats[0] / E_TOT
    var = stats[1] / E_TOT - (stats[0] / E_TOT) ** 2
    rvec = lax.rsqrt(var + EPS)
    return _edge_out(xg3, xpad, w1t, w1b, b1.reshape(1, 64),
                     g1.reshape(1, 64), mean.reshape(1, 64),
                     rvec.reshape(1, 64), be1.reshape(1, 64), w2,
                     b2.reshape(1, 64))


def _mlp_body(x1_ref, x2_ref, x3_ref, w1_ref, b1_ref, w2_ref, b2_ref,
              w3_ref, b3_ref, out_ref):
    w116 = w1_ref[...].astype(BF)
    h = (jnp.dot(x1_ref[...].astype(BF), w116[0:64],
                 preferred_element_type=jnp.float32)
         + jnp.dot(x2_ref[...].astype(BF), w116[64:128],
                   preferred_element_type=jnp.float32)
         + jnp.dot(x3_ref[...].astype(BF), w116[128:192],
                   preferred_element_type=jnp.float32))
    h = jnp.maximum(h + b1_ref[...], 0.0)
    h = jnp.dot(h.astype(BF), w2_ref[...].astype(BF),
                preferred_element_type=jnp.float32)
    h = jnp.maximum(h + b2_ref[...], 0.0)
    out_ref[...] = jnp.dot(h.astype(BF), w3_ref[...].astype(BF),
                           preferred_element_type=jnp.float32) + b3_ref[...]


def _final_mlp(x1, x2, x3, w1, b1, w2, b2, w3, b3):
    return pl.pallas_call(
        _mlp_body,
        grid=(B,),
        in_specs=[
            pl.BlockSpec((P, 64), lambda g: (g, 0)),
            pl.BlockSpec((P, 64), lambda g: (g, 0)),
            pl.BlockSpec((P, 64), lambda g: (g, 0)),
            pl.BlockSpec((192, 1024), lambda g: (0, 0)),
            pl.BlockSpec((1, 1024), lambda g: (0, 0)),
            pl.BlockSpec((1024, 512), lambda g: (0, 0)),
            pl.BlockSpec((1, 512), lambda g: (0, 0)),
            pl.BlockSpec((512, OUT), lambda g: (0, 0)),
            pl.BlockSpec((1, OUT), lambda g: (0, 0)),
        ],
        out_specs=pl.BlockSpec((P, OUT), lambda g: (g, 0)),
        out_shape=jax.ShapeDtypeStruct((N, OUT), jnp.float32),
    )(x1, x2, x3, w1, b1.reshape(1, 1024), w2, b2.reshape(1, 512),
      w3, b3.reshape(1, OUT))


def kernel(x, pos, batch, batch_size,
           c1_W1, c1_b1, c1_g1, c1_be1, c1_W2, c1_b2,
           c2_W1, c2_b1, c2_g1, c2_be1, c2_W2, c2_b2,
           c3_W1, c3_b1, c3_g1, c3_be1, c3_W2, c3_b2,
           m_W1, m_b1, m_W2, m_b2, m_W3, m_b3):
    x0 = jnp.concatenate([x, pos], axis=-1)
    x1 = _edge_conv(x0, c1_W1, c1_b1, c1_g1, c1_be1, c1_W2, c1_b2)
    x2 = _edge_conv(x1, c2_W1, c2_b1, c2_g1, c2_be1, c2_W2, c2_b2)
    x3 = _edge_conv(x2, c3_W1, c3_b1, c3_g1, c3_be1, c3_W2, c3_b2)
    out = _final_mlp(x1, x2, x3, m_W1, m_b1, m_W2, m_b2, m_W3, m_b3)
    return out.reshape(B, P, OUT)


# speedup vs baseline: 12.2408x; 1.7791x over previous
import functools

import jax
import jax.numpy as jnp
from jax import lax
from jax.experimental import pallas as pl
from jax.experimental.pallas import tpu as pltpu
from jax.experimental.pallas import tpu_sc as plsc

B = 8
P = 1024
N = B * P
K = 30
KPAD = 32
OUT = 13
EPS = 1e-5
SPLITS = 2
RB = P // SPLITS
E_TOT = float(N * K)
BF = jnp.bfloat16

NC = 2
NS = 16
NW = NC * NS
B_PER_W = (N * K) // NW
CH = 128
NCH = B_PER_W // CH
NBUF = 6


def _knn_body(xr_ref, xc_ref, idx_ref):
    b = pl.program_id(0)
    xr = xr_ref[...]
    xc = xc_ref[...]
    sqr = jnp.sum(xr * xr, axis=1, keepdims=True)
    sqc = jnp.sum(xc * xc, axis=1)
    gram = lax.dot_general(xr.astype(BF), xc.astype(BF),
                           (((1,), (1,)), ((), ())),
                           preferred_element_type=jnp.float32)
    d = (sqr - 2.0 * gram) + sqc[None, :]

    iota = lax.broadcasted_iota(jnp.int32, (RB, P), 1).astype(jnp.float32)
    for k in range(K):
        m = jnp.min(d, axis=1, keepdims=True)
        cand = jnp.where(d == m, iota, jnp.float32(P))
        j = jnp.min(cand, axis=1, keepdims=True)
        d = jnp.where(iota == j, jnp.inf, d)
        idx_ref[:, k:k + 1] = j.astype(jnp.int32) + b * P


def _knn(x):
    c = x.shape[1]
    return pl.pallas_call(
        _knn_body,
        grid=(B, SPLITS),
        in_specs=[
            pl.BlockSpec((RB, c), lambda b, h: (b * SPLITS + h, 0)),
            pl.BlockSpec((P, c), lambda b, h: (b, 0)),
        ],
        out_specs=pl.BlockSpec((RB, KPAD), lambda b, h: (b * SPLITS + h, 0)),
        out_shape=jax.ShapeDtypeStruct((N, KPAD), jnp.int32),
    )(x, x)


def _sc_gather_body(cw, table_ref, idx_ref, out_ref, idx_v, bufs, *sems):
    gsem = sems[:NBUF]
    ssem = sems[NBUF:]
    wid = lax.axis_index("s") * NC + lax.axis_index("c")
    base = wid * B_PER_W
    pltpu.sync_copy(idx_ref.at[pl.ds(base, B_PER_W)], idx_v)
    for db in range(NBUF):
        pltpu.async_copy(table_ref.at[idx_v.at[pl.ds(db * CH, CH)]],
                         bufs.at[db], gsem[db])

    def outer(it, carry):
        ci0 = it * NBUF
        for db in range(NBUF):
            ci = ci0 + db
            pltpu.make_async_copy(table_ref.at[pl.ds(0, CH)],
                                  bufs.at[db], gsem[db]).wait()
            pltpu.async_copy(bufs.at[db],
                             out_ref.at[pl.ds(base + ci * CH, CH)], ssem[db])
            nci = ci + NBUF

            @pl.when(nci < NCH)
            def _():
                pltpu.make_async_copy(bufs.at[db],
                                      out_ref.at[pl.ds(base, CH)],
                                      ssem[db]).wait()
                pltpu.async_copy(
                    table_ref.at[idx_v.at[pl.ds(nci * CH, CH)]],
                    bufs.at[db], gsem[db])
        return carry

    lax.fori_loop(0, NCH // NBUF, outer, 0)
    for db in range(NBUF):
        pltpu.make_async_copy(bufs.at[db],
                              out_ref.at[pl.ds(base, CH)], ssem[db]).wait()


def _sc_gather(table, idx_flat):
    cw = table.shape[1]
    kfn = functools.partial(
        pl.kernel,
        mesh=plsc.VectorSubcoreMesh(core_axis_name="c", subcore_axis_name="s"),
        compiler_params=pltpu.CompilerParams(use_tc_tiling_on_sc=False),
        out_type=jax.ShapeDtypeStruct((N * K, cw), jnp.float32),
        scratch_types=(
            [pltpu.VMEM((B_PER_W,), jnp.int32),
             pltpu.VMEM((NBUF, CH, cw), jnp.float32)]
            + [pltpu.SemaphoreType.DMA] * (2 * NBUF)
        ),
    )(functools.partial(_sc_gather_body, cw))
    return kfn(table, idx_flat)


PBB = 512


def _edge_h(xg_ref, xi_ref, w1t16, w1b16, b1):
    xi = xi_ref[...]
    hxi1 = jnp.dot(xi.astype(BF), w1t16,
                   preferred_element_type=jnp.float32) + b1
    for k in range(K):
        diff = xg_ref[k] - xi
        yield hxi1 + jnp.dot(diff.astype(BF), w1b16,
                             preferred_element_type=jnp.float32)


def _moments_body(xg_ref, xi_ref, w1t_ref, w1b_ref, b1_ref, stats_ref):
    w1t16 = w1t_ref[...].astype(BF)
    w1b16 = w1b_ref[...].astype(BF)
    b1 = b1_ref[...]
    hs = [jnp.zeros((PBB, 64), jnp.float32) for _ in range(2)]
    h2 = [jnp.zeros((PBB, 64), jnp.float32) for _ in range(2)]
    for k, h in enumerate(_edge_h(xg_ref, xi_ref, w1t16, w1b16, b1)):
        hs[k % 2] = hs[k % 2] + h
        h2[k % 2] = h2[k % 2] + h * h
    s1 = jnp.sum(hs[0] + hs[1], axis=0, keepdims=True)
    s2 = jnp.sum(h2[0] + h2[1], axis=0, keepdims=True)

    @pl.when(pl.program_id(0) == 0)
    def _():
        stats_ref[...] = jnp.zeros_like(stats_ref)

    stats_ref[0:1, :] += s1
    stats_ref[1:2, :] += s2


def _moments(xg3, xpad, w1t, w1b, b1):
    cw = xpad.shape[1]
    return pl.pallas_call(
        _moments_body,
        grid=(N // PBB,),
        in_specs=[
            pl.BlockSpec((K, PBB, cw), lambda g: (0, g, 0)),
            pl.BlockSpec((PBB, cw), lambda g: (g, 0)),
            pl.BlockSpec((cw, 64), lambda g: (0, 0)),
            pl.BlockSpec((cw, 64), lambda g: (0, 0)),
            pl.BlockSpec((1, 64), lambda g: (0, 0)),
        ],
        out_specs=pl.BlockSpec((8, 64), lambda g: (0, 0)),
        out_shape=jax.ShapeDtypeStruct((8, 64), jnp.float32),
    )(xg3, xpad, w1t, w1b, b1)


def _edge_out_body(xg_ref, xi_ref, w1t_ref, w1b_ref, b1_ref,
                   a_ref, c_ref, w2_ref, b2_ref, out_ref):
    w1t16 = w1t_ref[...].astype(BF)
    w1b16 = w1b_ref[...].astype(BF)
    w216 = w2_ref[...].astype(BF)
    b1 = b1_ref[...]
    av = a_ref[...]
    cv = c_ref[...]
    acc = [jnp.full((PBB, 64), -jnp.inf, jnp.float32) for _ in range(2)]
    for k, h in enumerate(_edge_h(xg_ref, xi_ref, w1t16, w1b16, b1)):
        t = jnp.maximum(h * av + cv, 0.0)
        o = jnp.dot(t.astype(BF), w216, preferred_element_type=jnp.float32)
        acc[k % 2] = jnp.maximum(acc[k % 2], o)
    out_ref[...] = jnp.maximum(acc[0], acc[1]) + b2_ref[...]


def _edge_out(xg3, xpad, w1t, w1b, b1, avec, cvec, w2, b2):
    cw = xpad.shape[1]
    return pl.pallas_call(
        _edge_out_body,
        grid=(N // PBB,),
        in_specs=[
            pl.BlockSpec((K, PBB, cw), lambda g: (0, g, 0)),
            pl.BlockSpec((PBB, cw), lambda g: (g, 0)),
            pl.BlockSpec((cw, 64), lambda g: (0, 0)),
            pl.BlockSpec((cw, 64), lambda g: (0, 0)),
            pl.BlockSpec((1, 64), lambda g: (0, 0)),
            pl.BlockSpec((1, 64), lambda g: (0, 0)),
            pl.BlockSpec((1, 64), lambda g: (0, 0)),
            pl.BlockSpec((64, 64), lambda g: (0, 0)),
            pl.BlockSpec((1, 64), lambda g: (0, 0)),
        ],
        out_specs=pl.BlockSpec((PBB, 64), lambda g: (g, 0)),
        out_shape=jax.ShapeDtypeStruct((N, 64), jnp.float32),
    )(xg3, xpad, w1t, w1b, b1, avec, cvec, w2, b2)


def _edge_conv(x, w1, b1, g1, be1, w2, b2):
    c = x.shape[1]
    cw = 16 if c < 16 else c
    if cw != c:
        xpad = jnp.zeros((N, cw), jnp.float32).at[:, :c].set(x)
        w1t = jnp.zeros((cw, 64), jnp.float32).at[:c].set(w1[:c])
        w1b = jnp.zeros((cw, 64), jnp.float32).at[:c].set(w1[c:])
    else:
        xpad = x
        w1t = w1[:c]
        w1b = w1[c:]
    idx = _knn(x)
    idx_flat = idx[:, :K].T.reshape(N * K)
    xg3 = _sc_gather(xpad, idx_flat).reshape(K, N, cw)
    stats = _moments(xg3, xpad, w1t, w1b, b1.reshape(1, 64))
    mean = stats[0] / E_TOT
    var = stats[1] / E_TOT - (stats[0] / E_TOT) ** 2
    rvec = lax.rsqrt(var + EPS)
    avec = g1 * rvec
    cvec = be1 - mean * g1 * rvec
    return _edge_out(xg3, xpad, w1t, w1b, b1.reshape(1, 64),
                     avec.reshape(1, 64), cvec.reshape(1, 64), w2,
                     b2.reshape(1, 64))


def _mlp_body(x1_ref, x2_ref, x3_ref, w1_ref, b1_ref, w2_ref, b2_ref,
              w3_ref, b3_ref, out_ref):
    w116 = w1_ref[...].astype(BF)
    h = (jnp.dot(x1_ref[...].astype(BF), w116[0:64],
                 preferred_element_type=jnp.float32)
         + jnp.dot(x2_ref[...].astype(BF), w116[64:128],
                   preferred_element_type=jnp.float32)
         + jnp.dot(x3_ref[...].astype(BF), w116[128:192],
                   preferred_element_type=jnp.float32))
    h = jnp.maximum(h + b1_ref[...], 0.0)
    h = jnp.dot(h.astype(BF), w2_ref[...].astype(BF),
                preferred_element_type=jnp.float32)
    h = jnp.maximum(h + b2_ref[...], 0.0)
    out_ref[...] = jnp.dot(h.astype(BF), w3_ref[...].astype(BF),
                           preferred_element_type=jnp.float32) + b3_ref[...]


def _final_mlp(x1, x2, x3, w1, b1, w2, b2, w3, b3):
    return pl.pallas_call(
        _mlp_body,
        grid=(B,),
        in_specs=[
            pl.BlockSpec((P, 64), lambda g: (g, 0)),
            pl.BlockSpec((P, 64), lambda g: (g, 0)),
            pl.BlockSpec((P, 64), lambda g: (g, 0)),
            pl.BlockSpec((192, 1024), lambda g: (0, 0)),
            pl.BlockSpec((1, 1024), lambda g: (0, 0)),
            pl.BlockSpec((1024, 512), lambda g: (0, 0)),
            pl.BlockSpec((1, 512), lambda g: (0, 0)),
            pl.BlockSpec((512, OUT), lambda g: (0, 0)),
            pl.BlockSpec((1, OUT), lambda g: (0, 0)),
        ],
        out_specs=pl.BlockSpec((P, OUT), lambda g: (g, 0)),
        out_shape=jax.ShapeDtypeStruct((N, OUT), jnp.float32),
    )(x1, x2, x3, w1, b1.reshape(1, 1024), w2, b2.reshape(1, 512),
      w3, b3.reshape(1, OUT))


def kernel(x, pos, batch, batch_size,
           c1_W1, c1_b1, c1_g1, c1_be1, c1_W2, c1_b2,
           c2_W1, c2_b1, c2_g1, c2_be1, c2_W2, c2_b2,
           c3_W1, c3_b1, c3_g1, c3_be1, c3_W2, c3_b2,
           m_W1, m_b1, m_W2, m_b2, m_W3, m_b3):
    x0 = jnp.concatenate([x, pos], axis=-1)
    x1 = _edge_conv(x0, c1_W1, c1_b1, c1_g1, c1_be1, c1_W2, c1_b2)
    x2 = _edge_conv(x1, c2_W1, c2_b1, c2_g1, c2_be1, c2_W2, c2_b2)
    x3 = _edge_conv(x2, c3_W1, c3_b1, c3_g1, c3_be1, c3_W2, c3_b2)
    out = _final_mlp(x1, x2, x3, m_W1, m_b1, m_W2, m_b2, m_W3, m_b3)
    return out.reshape(B, P, OUT)
